# baseline (device time: 113233 ns/iter reference)
import functools

import jax
import jax.numpy as jnp
from jax import lax
from jax.experimental import pallas as pl
from jax.experimental.pallas import tpu as pltpu

N_DEV = 4
B = 8
H = 8
D = 128
BS = 16
NB = 512
PP = 512
KK = PP * BS
NEG = -1e30


def _qk_body(q_ref, k_ref, bt_ref, lens_ref, p_ref, m_ref, l_ref,
             ckeys_scr):
    my = lax.axis_index("i")
    off = my * PP

    @pl.when(pl.program_id(0) == 0)
    def _():
        bt = bt_ref[...]
        lens = lens_ref[...]
        JC = 128
        c = jnp.zeros((B, PP), jnp.float32)
        for j0 in range(0, NB, JC):
            btc = bt[:, j0:j0 + JC]
            jio = lax.broadcasted_iota(jnp.int32, (B, JC, PP), 1) + j0
            pio = lax.broadcasted_iota(jnp.int32, (B, JC, PP), 2)
            hitc = jnp.where(
                (btc[:, :, None] == pio + off) & (jio < lens[:, :, None]),
                1.0, 0.0,
            )
            c = c + jnp.sum(hitc, axis=1)

        prow = lax.broadcasted_iota(jnp.int32, (PP, KK), 0)
        kcol = lax.broadcasted_iota(jnp.int32, (PP, KK), 1)
        expand = jnp.where(prow == kcol // BS, 1.0, 0.0).astype(jnp.bfloat16)
        ckeys_scr[...] = lax.dot_general(
            c.astype(jnp.bfloat16), expand, (((1,), (0,)), ((), ())),
            preferred_element_type=jnp.float32,
        )

    ck = ckeys_scr[...]
    q_h = q_ref[0]
    k_h = k_ref[...].reshape(KK, D)
    s_h = lax.dot_general(
        q_h, k_h, (((1,), (1,)), ((), ())),
        preferred_element_type=jnp.float32,
    ) * (D ** -0.5)
    sm = jnp.where(ck > 0.0, s_h, NEG)
    m = jnp.max(sm, axis=1, keepdims=True)
    p = jnp.exp(sm - m) * ck
    p_ref[0] = p
    m_ref[0] = m
    l_ref[0] = jnp.sum(p, axis=1, keepdims=True)


def _pv_body(p_ref, m_ref, l_ref, v_ref, out_ref,
             o_comm, m_comm, l_comm, send_sems, recv_sems):
    my = lax.axis_index("i")
    h = pl.program_id(0)

    v_h = v_ref[...].reshape(KK, D)
    o_comm[my, h] = lax.dot_general(
        p_ref[0], v_h, (((1,), (0,)), ((), ())),
        preferred_element_type=jnp.float32,
    )

    @pl.when(h == H - 1)
    def _():
        m_comm[my] = m_ref[...]
        l_comm[my] = l_ref[...]

        bar = pltpu.get_barrier_semaphore()
        for dlt in range(1, N_DEV):
            tgt = lax.rem(my + dlt, N_DEV)
            pl.semaphore_signal(bar, inc=1, device_id=(tgt,),
                                device_id_type=pl.DeviceIdType.MESH)
        pl.semaphore_wait(bar, N_DEV - 1)

        sends = []
        for dlt in range(1, N_DEV):
            tgt = lax.rem(my + dlt, N_DEV)
            for t, buf in ((0, o_comm), (1, m_comm), (2, l_comm)):
                r = pltpu.make_async_remote_copy(
                    src_ref=buf.at[my], dst_ref=buf.at[my],
                    send_sem=send_sems.at[dlt - 1, t],
                    recv_sem=recv_sems.at[my, t],
                    device_id=(tgt,), device_id_type=pl.DeviceIdType.MESH,
                )
                r.start()
                sends.append(r)

        for dlt in range(1, N_DEV):
            src = lax.rem(my + dlt, N_DEV)
            for t, buf in ((0, o_comm), (1, m_comm), (2, l_comm)):
                rw = pltpu.make_async_remote_copy(
                    src_ref=buf.at[src], dst_ref=buf.at[src],
                    send_sem=send_sems.at[dlt - 1, t],
                    recv_sem=recv_sems.at[src, t],
                    device_id=(src,), device_id_type=pl.DeviceIdType.MESH,
                )
                rw.wait_recv()
        for r in sends:
            r.wait_send()

        mall = m_comm[...]
        lall = l_comm[...]
        mg = jnp.max(mall, axis=0, keepdims=True)
        alpha = jnp.exp(mall - mg)
        lg = jnp.sum(alpha * lall, axis=0)
        onum = jnp.sum(alpha * o_comm[...], axis=0)
        og = onum / lg
        out_ref[:, 0] = jnp.transpose(og, (1, 0, 2))

        @functools.partial(pl.run_scoped,
                           exit_sem=pltpu.SemaphoreType.REGULAR)
        def _(exit_sem):
            for dlt in range(1, N_DEV):
                tgt = lax.rem(my + dlt, N_DEV)
                pl.semaphore_signal(exit_sem, inc=1, device_id=(tgt,),
                                    device_id_type=pl.DeviceIdType.MESH)
            pl.semaphore_wait(exit_sem, N_DEV - 1)


def kernel(Q, K, V, bt, lens):
    lens2 = lens.reshape(B, 1)
    qh = jnp.transpose(Q[:, 0], (1, 0, 2))
    k2 = K.reshape(PP, BS, H * D)
    v2 = V.reshape(PP, BS, H * D)

    p_part, m_part, l_part = pl.pallas_call(
        _qk_body,
        grid=(H,),
        out_shape=[
            jax.ShapeDtypeStruct((H, B, KK), jnp.float32),
            jax.ShapeDtypeStruct((H, B, 1), jnp.float32),
            jax.ShapeDtypeStruct((H, B, 1), jnp.float32),
        ],
        in_specs=[
            pl.BlockSpec((1, B, D), lambda h: (h, 0, 0)),
            pl.BlockSpec((PP, BS, D), lambda h: (0, 0, h)),
            pl.BlockSpec((B, NB), lambda h: (0, 0)),
            pl.BlockSpec((B, 1), lambda h: (0, 0)),
        ],
        out_specs=[
            pl.BlockSpec((1, B, KK), lambda h: (h, 0, 0)),
            pl.BlockSpec((1, B, 1), lambda h: (h, 0, 0)),
            pl.BlockSpec((1, B, 1), lambda h: (h, 0, 0)),
        ],
        scratch_shapes=[pltpu.VMEM((B, KK), jnp.float32)],
        compiler_params=pltpu.CompilerParams(
            vmem_limit_bytes=60 * 1024 * 1024,
        ),
    )(qh, k2, bt, lens2)

    return pl.pallas_call(
        _pv_body,
        grid=(H,),
        out_shape=jax.ShapeDtypeStruct((B, 1, H, D), jnp.float32),
        in_specs=[
            pl.BlockSpec((1, B, KK), lambda h: (h, 0, 0)),
            pl.BlockSpec((H, B, 1), lambda h: (0, 0, 0)),
            pl.BlockSpec((H, B, 1), lambda h: (0, 0, 0)),
            pl.BlockSpec((PP, BS, D), lambda h: (0, 0, h)),
        ],
        out_specs=pl.BlockSpec((B, 1, H, D), lambda h: (0, 0, 0, 0)),
        scratch_shapes=[
            pltpu.VMEM((N_DEV, H, B, D), jnp.float32),
            pltpu.VMEM((N_DEV, H, B, 1), jnp.float32),
            pltpu.VMEM((N_DEV, H, B, 1), jnp.float32),
            pltpu.SemaphoreType.DMA((N_DEV - 1, 3)),
            pltpu.SemaphoreType.DMA((N_DEV, 3)),
        ],
        compiler_params=pltpu.CompilerParams(
            collective_id=0,
            vmem_limit_bytes=60 * 1024 * 1024,
        ),
    )(p_part, m_part, l_part, v2)


# device time: 108654 ns/iter; 1.0421x vs baseline; 1.0421x over previous
import functools

import jax
import jax.numpy as jnp
from jax import lax
from jax.experimental import pallas as pl
from jax.experimental.pallas import tpu as pltpu

N_DEV = 4
B = 8
H = 8
D = 128
BS = 16
NB = 512
PP = 512
KK = PP * BS
HD = H * D
R = B * H
PC = 64
CK = PC * BS
NC = PP // PC
NEG = -1e30


def _body(qbd_ref, k_ref, v_ref, bt_ref, lens_ref, out_ref,
          ck_scr, m_scr, l_scr, o_scr,
          o_comm, ml_comm, send_sems, recv_sems):
    my = lax.axis_index("i")
    c_id = pl.program_id(0)

    @pl.when(c_id == 0)
    def _init():
        off = my * PP
        bt = bt_ref[...]
        lens = lens_ref[...]
        JC = 128
        c = jnp.zeros((B, PP), jnp.float32)
        for j0 in range(0, NB, JC):
            btc = bt[:, j0:j0 + JC]
            jio = lax.broadcasted_iota(jnp.int32, (B, JC, PP), 1) + j0
            pio = lax.broadcasted_iota(jnp.int32, (B, JC, PP), 2)
            hitc = jnp.where(
                (btc[:, :, None] == pio + off) & (jio < lens[:, :, None]),
                1.0, 0.0,
            )
            c = c + jnp.sum(hitc, axis=1)

        prow = lax.broadcasted_iota(jnp.int32, (PP, KK), 0)
        kcol = lax.broadcasted_iota(jnp.int32, (PP, KK), 1)
        expand = jnp.where(prow == kcol // BS, 1.0, 0.0).astype(jnp.bfloat16)
        ckeys = lax.dot_general(
            c.astype(jnp.bfloat16), expand, (((1,), (0,)), ((), ())),
            preferred_element_type=jnp.float32,
        )
        ck_scr[...] = jnp.broadcast_to(
            ckeys[:, None, :], (B, H, KK)
        ).reshape(R, KK)

        m_scr[...] = jnp.full((R, 1), NEG, jnp.float32)
        l_scr[...] = jnp.zeros((R, 1), jnp.float32)
        o_scr[...] = jnp.zeros((R, HD), jnp.float32)

    qbd = qbd_ref[...]
    k_c = k_ref[...].reshape(CK, HD).astype(jnp.bfloat16)
    s_c = lax.dot_general(
        qbd, k_c, (((1,), (1,)), ((), ())),
        preferred_element_type=jnp.float32,
    ) * (D ** -0.5)
    ck_c = ck_scr[:, pl.ds(c_id * CK, CK)]
    sm = jnp.where(ck_c > 0.0, s_c, NEG)

    m_old = m_scr[...]
    m_new = jnp.maximum(m_old, jnp.max(sm, axis=1, keepdims=True))
    a = jnp.exp(m_old - m_new)
    p_c = jnp.exp(sm - m_new) * ck_c
    v_c = v_ref[...].reshape(CK, HD).astype(jnp.bfloat16)
    pv = lax.dot_general(
        p_c.astype(jnp.bfloat16), v_c, (((1,), (0,)), ((), ())),
        preferred_element_type=jnp.float32,
    )
    m_scr[...] = m_new
    l_scr[...] = l_scr[...] * a + jnp.sum(p_c, axis=1, keepdims=True)
    o_scr[...] = o_scr[...] * a + pv

    @pl.when(c_id == NC - 1)
    def _finish():
        rows = lax.broadcasted_iota(jnp.int32, (R, 1), 0)
        hrow = lax.rem(rows, H)
        o_full = o_scr[...]
        o_part = jnp.zeros((R, D), jnp.float32)
        for blk in range(H):
            sel = jnp.where(hrow == blk, 1.0, 0.0)
            o_part = o_part + o_full[:, blk * D:(blk + 1) * D] * sel

        o_comm[my] = o_part
        ml_comm[my] = jnp.concatenate(
            [m_scr[...], l_scr[...]], axis=1
        )

        bar = pltpu.get_barrier_semaphore()
        for dlt in range(1, N_DEV):
            tgt = lax.rem(my + dlt, N_DEV)
            pl.semaphore_signal(bar, inc=1, device_id=(tgt,),
                                device_id_type=pl.DeviceIdType.MESH)
        pl.semaphore_wait(bar, N_DEV - 1)

        sends = []
        for dlt in range(1, N_DEV):
            tgt = lax.rem(my + dlt, N_DEV)
            for t, buf in ((0, o_comm), (1, ml_comm)):
                r = pltpu.make_async_remote_copy(
                    src_ref=buf.at[my], dst_ref=buf.at[my],
                    send_sem=send_sems.at[dlt - 1, t],
                    recv_sem=recv_sems.at[my, t],
                    device_id=(tgt,), device_id_type=pl.DeviceIdType.MESH,
                )
                r.start()
                sends.append(r)

        for dlt in range(1, N_DEV):
            src = lax.rem(my + dlt, N_DEV)
            for t, buf in ((0, o_comm), (1, ml_comm)):
                rw = pltpu.make_async_remote_copy(
                    src_ref=buf.at[src], dst_ref=buf.at[src],
                    send_sem=send_sems.at[dlt - 1, t],
                    recv_sem=recv_sems.at[src, t],
                    device_id=(src,), device_id_type=pl.DeviceIdType.MESH,
                )
                rw.wait_recv()
        for r in sends:
            r.wait_send()

        mall = ml_comm[:, :, 0:1]
        lall = ml_comm[:, :, 1:2]
        mg = jnp.max(mall, axis=0, keepdims=True)
        alpha = jnp.exp(mall - mg)
        lg = jnp.sum(alpha * lall, axis=0)
        onum = jnp.sum(alpha * o_comm[...], axis=0)
        og = onum / lg
        out_ref[:, 0] = og.reshape(B, H, D)

        @functools.partial(pl.run_scoped,
                           exit_sem=pltpu.SemaphoreType.REGULAR)
        def _(exit_sem):
            for dlt in range(1, N_DEV):
                tgt = lax.rem(my + dlt, N_DEV)
                pl.semaphore_signal(exit_sem, inc=1, device_id=(tgt,),
                                    device_id_type=pl.DeviceIdType.MESH)
            pl.semaphore_wait(exit_sem, N_DEV - 1)


def kernel(Q, K, V, bt, lens):
    lens2 = lens.reshape(B, 1)
    eye = jnp.eye(H, dtype=jnp.float32)
    qbd = (Q[:, 0][:, :, None, :] * eye[None, :, :, None]).reshape(R, HD)
    qbd = qbd.astype(jnp.bfloat16)
    k2 = K.reshape(PP, BS, HD)
    v2 = V.reshape(PP, BS, HD)

    return pl.pallas_call(
        _body,
        grid=(NC,),
        out_shape=jax.ShapeDtypeStruct((B, 1, H, D), jnp.float32),
        in_specs=[
            pl.BlockSpec((R, HD), lambda c: (0, 0)),
            pl.BlockSpec((PC, BS, HD), lambda c: (c, 0, 0)),
            pl.BlockSpec((PC, BS, HD), lambda c: (c, 0, 0)),
            pl.BlockSpec((B, NB), lambda c: (0, 0)),
            pl.BlockSpec((B, 1), lambda c: (0, 0)),
        ],
        out_specs=pl.BlockSpec((B, 1, H, D), lambda c: (0, 0, 0, 0)),
        scratch_shapes=[
            pltpu.VMEM((R, KK), jnp.float32),
            pltpu.VMEM((R, 1), jnp.float32),
            pltpu.VMEM((R, 1), jnp.float32),
            pltpu.VMEM((R, HD), jnp.float32),
            pltpu.VMEM((N_DEV, R, D), jnp.float32),
            pltpu.VMEM((N_DEV, R, 2), jnp.float32),
            pltpu.SemaphoreType.DMA((N_DEV - 1, 2)),
            pltpu.SemaphoreType.DMA((N_DEV, 2)),
        ],
        compiler_params=pltpu.CompilerParams(
            collective_id=0,
            vmem_limit_bytes=60 * 1024 * 1024,
        ),
    )(qbd, k2, v2, bt, lens2)


# device time: 42931 ns/iter; 2.6376x vs baseline; 2.5309x over previous
import functools

import jax
import jax.numpy as jnp
from jax import lax
from jax.experimental import pallas as pl
from jax.experimental.pallas import tpu as pltpu

N_DEV = 4
B = 8
H = 8
D = 128
BS = 16
NB = 512
PP = 512
R = B * H
PC = 64
CC = PC * BS * H
NC = PP // PC
NEG = -1e30


def _body(q_ref, k_ref, v_ref, bt_ref, lens_ref, out_ref,
          c_scr, hmask_scr, exp2_scr, m_scr, l_scr, o_scr,
          o_comm, ml_comm, send_sems, recv_sems):
    my = lax.axis_index("i")
    c_id = pl.program_id(0)

    @pl.when(c_id == 0)
    def _init():
        off = my * PP
        bt = bt_ref[...]
        lens = lens_ref[...]
        JC = 128
        c = jnp.zeros((B, PP), jnp.float32)
        for j0 in range(0, NB, JC):
            btc = bt[:, j0:j0 + JC]
            jio = lax.broadcasted_iota(jnp.int32, (B, JC, PP), 1) + j0
            pio = lax.broadcasted_iota(jnp.int32, (B, JC, PP), 2)
            hitc = jnp.where(
                (btc[:, :, None] == pio + off) & (jio < lens[:, :, None]),
                1.0, 0.0,
            )
            c = c + jnp.sum(hitc, axis=1)
        for cc in range(NC):
            c_scr[cc] = c[:, cc * PC:(cc + 1) * PC]

        rio = lax.rem(lax.broadcasted_iota(jnp.int32, (R, CC), 0), H)
        cio = lax.rem(lax.broadcasted_iota(jnp.int32, (R, CC), 1), H)
        hmask_scr[...] = jnp.where(rio == cio, 1.0, 0.0)

        pro = lax.broadcasted_iota(jnp.int32, (PC, CC), 0)
        cco = lax.broadcasted_iota(jnp.int32, (PC, CC), 1)
        exp2_scr[...] = jnp.where(
            pro == cco // (BS * H), 1.0, 0.0
        ).astype(jnp.bfloat16)

        m_scr[...] = jnp.full((R, 1), NEG, jnp.float32)
        l_scr[...] = jnp.zeros((R, 1), jnp.float32)
        o_scr[...] = jnp.zeros((R, D), jnp.float32)

    k_c = k_ref[...].astype(jnp.bfloat16)
    s = lax.dot_general(
        q_ref[...], k_c, (((1,), (1,)), ((), ())),
        preferred_element_type=jnp.float32,
    ) * (D ** -0.5)

    ckx = lax.dot_general(
        c_scr[c_id].astype(jnp.bfloat16), exp2_scr[...],
        (((1,), (0,)), ((), ())),
        preferred_element_type=jnp.float32,
    )
    ckm = jnp.broadcast_to(ckx[:, None, :], (B, H, CC)).reshape(R, CC)
    ckm = ckm * hmask_scr[...]

    sm = jnp.where(ckm > 0.0, s, NEG)
    m_old = m_scr[...]
    m_new = jnp.maximum(m_old, jnp.max(sm, axis=1, keepdims=True))
    a = jnp.exp(m_old - m_new)
    p_c = jnp.exp(sm - m_new) * ckm
    pv = lax.dot_general(
        p_c.astype(jnp.bfloat16), v_ref[...].astype(jnp.bfloat16),
        (((1,), (0,)), ((), ())),
        preferred_element_type=jnp.float32,
    )
    m_scr[...] = m_new
    l_scr[...] = l_scr[...] * a + jnp.sum(p_c, axis=1, keepdims=True)
    o_scr[...] = o_scr[...] * a + pv

    @pl.when(c_id == NC - 1)
    def _finish():
        o_comm[my] = o_scr[...]
        ml_comm[my] = jnp.concatenate(
            [m_scr[...], l_scr[...]], axis=1
        )

        bar = pltpu.get_barrier_semaphore()
        for dlt in range(1, N_DEV):
            tgt = lax.rem(my + dlt, N_DEV)
            pl.semaphore_signal(bar, inc=1, device_id=(tgt,),
                                device_id_type=pl.DeviceIdType.MESH)
        pl.semaphore_wait(bar, N_DEV - 1)

        sends = []
        for dlt in range(1, N_DEV):
            tgt = lax.rem(my + dlt, N_DEV)
            for t, buf in ((0, o_comm), (1, ml_comm)):
                r = pltpu.make_async_remote_copy(
                    src_ref=buf.at[my], dst_ref=buf.at[my],
                    send_sem=send_sems.at[dlt - 1, t],
                    recv_sem=recv_sems.at[my, t],
                    device_id=(tgt,), device_id_type=pl.DeviceIdType.MESH,
                )
                r.start()
                sends.append(r)

        for dlt in range(1, N_DEV):
            src = lax.rem(my + dlt, N_DEV)
            for t, buf in ((0, o_comm), (1, ml_comm)):
                rw = pltpu.make_async_remote_copy(
                    src_ref=buf.at[src], dst_ref=buf.at[src],
                    send_sem=send_sems.at[dlt - 1, t],
                    recv_sem=recv_sems.at[src, t],
                    device_id=(src,), device_id_type=pl.DeviceIdType.MESH,
                )
                rw.wait_recv()
        for r in sends:
            r.wait_send()

        mall = ml_comm[:, :, 0:1]
        lall = ml_comm[:, :, 1:2]
        mg = jnp.max(mall, axis=0, keepdims=True)
        alpha = jnp.exp(mall - mg)
        lg = jnp.sum(alpha * lall, axis=0)
        onum = jnp.sum(alpha * o_comm[...], axis=0)
        og = onum / lg
        out_ref[:, 0] = og.reshape(B, H, D)

        @functools.partial(pl.run_scoped,
                           exit_sem=pltpu.SemaphoreType.REGULAR)
        def _(exit_sem):
            for dlt in range(1, N_DEV):
                tgt = lax.rem(my + dlt, N_DEV)
                pl.semaphore_signal(exit_sem, inc=1, device_id=(tgt,),
                                    device_id_type=pl.DeviceIdType.MESH)
            pl.semaphore_wait(exit_sem, N_DEV - 1)


def kernel(Q, K, V, bt, lens):
    lens2 = lens.reshape(B, 1)
    q2 = Q.reshape(R, D).astype(jnp.bfloat16)
    k2 = K.reshape(PP * BS * H, D)
    v2 = V.reshape(PP * BS * H, D)

    return pl.pallas_call(
        _body,
        grid=(NC,),
        out_shape=jax.ShapeDtypeStruct((B, 1, H, D), jnp.float32),
        in_specs=[
            pl.BlockSpec((R, D), lambda c: (0, 0)),
            pl.BlockSpec((CC, D), lambda c: (c, 0)),
            pl.BlockSpec((CC, D), lambda c: (c, 0)),
            pl.BlockSpec((B, NB), lambda c: (0, 0)),
            pl.BlockSpec((B, 1), lambda c: (0, 0)),
        ],
        out_specs=pl.BlockSpec((B, 1, H, D), lambda c: (0, 0, 0, 0)),
        scratch_shapes=[
            pltpu.VMEM((NC, B, PC), jnp.float32),
            pltpu.VMEM((R, CC), jnp.float32),
            pltpu.VMEM((PC, CC), jnp.bfloat16),
            pltpu.VMEM((R, 1), jnp.float32),
            pltpu.VMEM((R, 1), jnp.float32),
            pltpu.VMEM((R, D), jnp.float32),
            pltpu.VMEM((N_DEV, R, D), jnp.float32),
            pltpu.VMEM((N_DEV, R, 2), jnp.float32),
            pltpu.SemaphoreType.DMA((N_DEV - 1, 2)),
            pltpu.SemaphoreType.DMA((N_DEV, 2)),
        ],
        compiler_params=pltpu.CompilerParams(
            collective_id=0,
            vmem_limit_bytes=60 * 1024 * 1024,
        ),
    )(q2, k2, v2, bt, lens2)
